# baseline (speedup 1.0000x reference)
import functools

import jax
import jax.numpy as jnp
from jax import lax
from jax.experimental import pallas as pl
from jax.experimental.pallas import tpu as pltpu
from jax.experimental.pallas import tpu_sc as plsc

B, N, M, C, K = 16, 2048, 2048, 256, 3
TN = 512
NC, NS, L = 2, 16, 16
NW = NC * NS
SUB = 16
NCH = 2
CB = B // NCH


def _topk_body(boff, xyz_ref, cxyz_ref, idx_ref, w_ref):
    b = pl.program_id(0)
    x = xyz_ref[0]
    c = cxyz_ref[0]
    xb = x.astype(jnp.bfloat16).astype(jnp.float32)
    cb = c.astype(jnp.bfloat16).astype(jnp.float32)
    x2 = (x[0:1] * x[0:1] + x[1:2] * x[1:2]) + x[2:3] * x[2:3]
    c2 = (c[:, 0:1] * c[:, 0:1] + c[:, 1:2] * c[:, 1:2]) + c[:, 2:3] * c[:, 2:3]
    dot = cb[:, 0:1] * xb[0:1]
    dot = dot + cb[:, 1:2] * xb[1:2]
    dot = dot + cb[:, 2:3] * xb[2:3]
    sq = (x2 + c2) - 2.0 * dot
    cur = jnp.sqrt(jnp.maximum(sq, 1e-12))
    iota = lax.broadcasted_iota(jnp.int32, (M, TN), 0)
    ds, js = [], []
    for t in range(K):
        m = jnp.min(cur, axis=0, keepdims=True)
        i = jnp.min(jnp.where(cur == m, iota, M), axis=0, keepdims=True)
        ds.append(m)
        js.append(i)
        if t < K - 1:
            cur = jnp.where(iota == i, jnp.float32(3.0e38), cur)
    d = jnp.concatenate(ds, axis=0)
    w = 1.0 / jnp.maximum(d, 1e-8)
    wn = w / jnp.sum(w, axis=0, keepdims=True)
    idx_ref[...] = jnp.concatenate(js, axis=0) + (b + boff) * M
    w_ref[...] = wn


def _nearest_tc(xyz_t, cxyz, boff):
    nt = N // TN
    pc = CB * N
    return pl.pallas_call(
        functools.partial(_topk_body, boff),
        grid=(CB, nt),
        in_specs=[
            pl.BlockSpec((1, 3, TN), lambda b, n: (b, 0, n)),
            pl.BlockSpec((1, M, 3), lambda b, n: (b, 0, 0)),
        ],
        out_specs=[
            pl.BlockSpec((K, TN), lambda b, n: (0, b * nt + n)),
            pl.BlockSpec((K, TN), lambda b, n: (0, b * nt + n)),
        ],
        out_shape=[
            jax.ShapeDtypeStruct((K, pc), jnp.int32),
            jax.ShapeDtypeStruct((K, pc), jnp.float32),
        ],
    )(xyz_t, cxyz)


def _interp_body(table_hbm, idx_hbm, w_hbm, out_hbm,
                 idx_v, w_v, rows0, rows1, o0, o1,
                 sem_g0, sem_g1, sem_o0, sem_o1):
    pc = CB * N
    pw = pc // NW
    nsub = pw // SUB
    wid = lax.axis_index("s") * NC + lax.axis_index("c")
    base = pl.multiple_of(wid * pw, pw)
    for j in range(K):
        pltpu.sync_copy(idx_hbm.at[pl.ds(j * pc + base, pw)], idx_v.at[pl.ds(j * pw, pw)])
        pltpu.sync_copy(w_hbm.at[pl.ds(j * pc + base, pw)], w_v.at[pl.ds(j * pw, pw)])

    def issue_gather(s, rows, sem):
        off = pl.multiple_of(s * SUB, SUB)
        for j in range(K):
            pltpu.async_copy(
                table_hbm.at[idx_v.at[pl.ds(j * pw + off, SUB)]],
                rows.at[pl.ds(j * SUB, SUB)], sem)

    def wait_gather(rows, sem):
        pltpu.make_async_copy(table_hbm.at[pl.ds(0, K * SUB)], rows, sem).wait()

    def wait_out(o, sem):
        pltpu.make_async_copy(o, out_hbm.at[pl.ds(base, SUB)], sem).wait()

    def issue_out(s, o, sem):
        off = pl.multiple_of(s * SUB, SUB)
        pltpu.async_copy(o, out_hbm.at[pl.ds(base + off, SUB)], sem)

    def compute(s, rows, o):
        off = pl.multiple_of(s * SUB, SUB)
        wvs = [w_v[pl.ds(j * pw + off, L)] for j in range(K)]
        dn = lax.GatherDimensionNumbers(offset_dims=(), collapsed_slice_dims=(0,),
                                        start_index_map=(0,))

        @pl.loop(0, SUB)
        def _pt(pidx):
            lane = jnp.full((L, 1), pidx, jnp.int32)
            w0, w1, w2 = (lax.gather(wv, lane, dn, slice_sizes=(1,),
                                     mode=lax.GatherScatterMode.PROMISE_IN_BOUNDS)
                          for wv in wvs)
            for cc in range(C // L):
                sl = pl.ds(cc * L, L)
                r0 = rows[pidx, sl]
                r1 = rows[SUB + pidx, sl]
                r2 = rows[2 * SUB + pidx, sl]
                o[pidx, sl] = r0 * w0 + r1 * w1 + r2 * w2

    issue_gather(0, rows0, sem_g0)

    @pl.loop(0, nsub, step=2)
    def _ring(s0):
        wait_gather(rows0, sem_g0)
        issue_gather(s0 + 1, rows1, sem_g1)

        @pl.when(s0 >= 2)
        def _():
            wait_out(o0, sem_o0)

        compute(s0, rows0, o0)
        issue_out(s0, o0, sem_o0)

        wait_gather(rows1, sem_g1)

        @pl.when(s0 + 2 < nsub)
        def _():
            issue_gather(s0 + 2, rows0, sem_g0)

        @pl.when(s0 >= 2)
        def _():
            wait_out(o1, sem_o1)

        compute(s0 + 1, rows1, o1)
        issue_out(s0 + 1, o1, sem_o1)

    wait_out(o0, sem_o0)
    wait_out(o1, sem_o1)


@functools.cache
def _interp_sc():
    pc = CB * N
    pw = pc // NW
    return pl.kernel(
        _interp_body,
        out_type=jax.ShapeDtypeStruct((pc, C), jnp.float32),
        mesh=plsc.VectorSubcoreMesh(core_axis_name="c", subcore_axis_name="s",
                                    num_cores=NC, num_subcores=NS),
        scratch_types=[
            pltpu.VMEM((K * pw,), jnp.int32),
            pltpu.VMEM((K * pw,), jnp.float32),
            pltpu.VMEM((K * SUB, C), jnp.float32),
            pltpu.VMEM((K * SUB, C), jnp.float32),
            pltpu.VMEM((SUB, C), jnp.float32),
            pltpu.VMEM((SUB, C), jnp.float32),
            pltpu.SemaphoreType.DMA,
            pltpu.SemaphoreType.DMA,
            pltpu.SemaphoreType.DMA,
            pltpu.SemaphoreType.DMA,
        ],
    )


def kernel(p, f):
    xyz_t = jnp.transpose(p[0], (0, 2, 1))
    cxyz = p[1]
    table = jnp.transpose(f[0], (0, 2, 1)).reshape(B * M, C)
    pc = CB * N
    outs = []
    for ch in range(NCH):
        bs = slice(ch * CB, (ch + 1) * CB)
        idx, w = _nearest_tc(xyz_t[bs], cxyz[bs], ch * CB)
        outs.append(_interp_sc()(table, idx.reshape(K * pc), w.reshape(K * pc)))
    return jnp.concatenate(outs, axis=0).reshape(B, N, C)


# speedup vs baseline: 1.2198x; 1.2198x over previous
import functools

import jax
import jax.numpy as jnp
from jax import lax
from jax.experimental import pallas as pl
from jax.experimental.pallas import tpu as pltpu
from jax.experimental.pallas import tpu_sc as plsc

B, N, M, C, K = 16, 2048, 2048, 256, 3
TN = 1024
NC, NS, L = 2, 16, 16
NW = NC * NS
SUB = 16
NCH = 2
CB = B // NCH


def _topk_body(boff, xyz_ref, cxyz_ref, idx_ref, w_ref):
    b = pl.program_id(0)
    x = xyz_ref[0]
    c = cxyz_ref[0]
    xb = x.astype(jnp.bfloat16).astype(jnp.float32)
    cb = c.astype(jnp.bfloat16).astype(jnp.float32)
    x2 = (x[0:1] * x[0:1] + x[1:2] * x[1:2]) + x[2:3] * x[2:3]
    c2 = (c[:, 0:1] * c[:, 0:1] + c[:, 1:2] * c[:, 1:2]) + c[:, 2:3] * c[:, 2:3]
    dot = cb[:, 0:1] * xb[0:1]
    dot = dot + cb[:, 1:2] * xb[1:2]
    dot = dot + cb[:, 2:3] * xb[2:3]
    cur = jnp.maximum((x2 + c2) - 2.0 * dot, 1e-12)
    iota = lax.broadcasted_iota(jnp.int32, (M, TN), 0)
    ds, js = [], []
    for t in range(K):
        m = jnp.min(cur, axis=0, keepdims=True)
        i = jnp.min(jnp.where(cur == m, iota, M), axis=0, keepdims=True)
        ds.append(m)
        js.append(i)
        if t < K - 1:
            cur = jnp.where(iota == i, jnp.float32(3.0e38), cur)
    d = jnp.sqrt(jnp.concatenate(ds, axis=0))
    w = 1.0 / jnp.maximum(d, 1e-8)
    wn = w / jnp.sum(w, axis=0, keepdims=True)
    idx_ref[...] = jnp.concatenate(js, axis=0) + (b + boff) * M
    w_ref[...] = wn


def _nearest_tc(xyz_t, cxyz, boff):
    nt = N // TN
    pc = CB * N
    return pl.pallas_call(
        functools.partial(_topk_body, boff),
        grid=(CB, nt),
        in_specs=[
            pl.BlockSpec((1, 3, TN), lambda b, n: (b, 0, n)),
            pl.BlockSpec((1, M, 3), lambda b, n: (b, 0, 0)),
        ],
        out_specs=[
            pl.BlockSpec((K, TN), lambda b, n: (0, b * nt + n)),
            pl.BlockSpec((K, TN), lambda b, n: (0, b * nt + n)),
        ],
        out_shape=[
            jax.ShapeDtypeStruct((K, pc), jnp.int32),
            jax.ShapeDtypeStruct((K, pc), jnp.float32),
        ],
    )(xyz_t, cxyz)


def _interp_body(table_hbm, idx_hbm, w_hbm, out_hbm,
                 idx_v, w_v, rows0, rows1, o0, o1,
                 sem_g0, sem_g1, sem_o0, sem_o1):
    pc = CB * N
    pw = pc // NW
    nsub = pw // SUB
    wid = lax.axis_index("s") * NC + lax.axis_index("c")
    base = pl.multiple_of(wid * pw, pw)
    for j in range(K):
        pltpu.sync_copy(idx_hbm.at[pl.ds(j * pc + base, pw)], idx_v.at[pl.ds(j * pw, pw)])
        pltpu.sync_copy(w_hbm.at[pl.ds(j * pc + base, pw)], w_v.at[pl.ds(j * pw, pw)])

    def issue_gather(s, rows, sem):
        off = pl.multiple_of(s * SUB, SUB)
        for j in range(K):
            pltpu.async_copy(
                table_hbm.at[idx_v.at[pl.ds(j * pw + off, SUB)]],
                rows.at[pl.ds(j * SUB, SUB)], sem)

    def wait_gather(rows, sem):
        pltpu.make_async_copy(table_hbm.at[pl.ds(0, K * SUB)], rows, sem).wait()

    def wait_out(o, sem):
        pltpu.make_async_copy(o, out_hbm.at[pl.ds(base, SUB)], sem).wait()

    def issue_out(s, o, sem):
        off = pl.multiple_of(s * SUB, SUB)
        pltpu.async_copy(o, out_hbm.at[pl.ds(base + off, SUB)], sem)

    def compute(s, rows, o):
        off = pl.multiple_of(s * SUB, SUB)
        wvs = [w_v[pl.ds(j * pw + off, L)] for j in range(K)]
        dn = lax.GatherDimensionNumbers(offset_dims=(), collapsed_slice_dims=(0,),
                                        start_index_map=(0,))

        @pl.loop(0, SUB)
        def _pt(pidx):
            lane = jnp.full((L, 1), pidx, jnp.int32)
            w0, w1, w2 = (lax.gather(wv, lane, dn, slice_sizes=(1,),
                                     mode=lax.GatherScatterMode.PROMISE_IN_BOUNDS)
                          for wv in wvs)
            for cc in range(C // L):
                sl = pl.ds(cc * L, L)
                r0 = rows[pidx, sl]
                r1 = rows[SUB + pidx, sl]
                r2 = rows[2 * SUB + pidx, sl]
                o[pidx, sl] = r0 * w0 + r1 * w1 + r2 * w2

    issue_gather(0, rows0, sem_g0)

    @pl.loop(0, nsub, step=2)
    def _ring(s0):
        wait_gather(rows0, sem_g0)
        issue_gather(s0 + 1, rows1, sem_g1)

        @pl.when(s0 >= 2)
        def _():
            wait_out(o0, sem_o0)

        compute(s0, rows0, o0)
        issue_out(s0, o0, sem_o0)

        wait_gather(rows1, sem_g1)

        @pl.when(s0 + 2 < nsub)
        def _():
            issue_gather(s0 + 2, rows0, sem_g0)

        @pl.when(s0 >= 2)
        def _():
            wait_out(o1, sem_o1)

        compute(s0 + 1, rows1, o1)
        issue_out(s0 + 1, o1, sem_o1)

    wait_out(o0, sem_o0)
    wait_out(o1, sem_o1)


@functools.cache
def _interp_sc():
    pc = CB * N
    pw = pc // NW
    return pl.kernel(
        _interp_body,
        out_type=jax.ShapeDtypeStruct((pc, C), jnp.float32),
        mesh=plsc.VectorSubcoreMesh(core_axis_name="c", subcore_axis_name="s",
                                    num_cores=NC, num_subcores=NS),
        scratch_types=[
            pltpu.VMEM((K * pw,), jnp.int32),
            pltpu.VMEM((K * pw,), jnp.float32),
            pltpu.VMEM((K * SUB, C), jnp.float32),
            pltpu.VMEM((K * SUB, C), jnp.float32),
            pltpu.VMEM((SUB, C), jnp.float32),
            pltpu.VMEM((SUB, C), jnp.float32),
            pltpu.SemaphoreType.DMA,
            pltpu.SemaphoreType.DMA,
            pltpu.SemaphoreType.DMA,
            pltpu.SemaphoreType.DMA,
        ],
    )


def kernel(p, f):
    xyz_t = jnp.transpose(p[0], (0, 2, 1))
    cxyz = p[1]
    table = jnp.transpose(f[0], (0, 2, 1)).reshape(B * M, C)
    pc = CB * N
    outs = []
    for ch in range(NCH):
        bs = slice(ch * CB, (ch + 1) * CB)
        idx, w = _nearest_tc(xyz_t[bs], cxyz[bs], ch * CB)
        outs.append(_interp_sc()(table, idx.reshape(K * pc), w.reshape(K * pc)))
    return jnp.concatenate(outs, axis=0).reshape(B, N, C)


# speedup vs baseline: 1.2379x; 1.0148x over previous
import functools

import jax
import jax.numpy as jnp
from jax import lax
from jax.experimental import pallas as pl
from jax.experimental.pallas import tpu as pltpu
from jax.experimental.pallas import tpu_sc as plsc

B, N, M, C, K = 16, 2048, 2048, 256, 3
TN = 1024
NC, NS, L = 2, 16, 16
NW = NC * NS
SUB = 16
NCH = 1
CB = B // NCH


def _topk_body(boff, xyz_ref, cxyz_ref, f_ref, idx_ref, w_ref, ft_ref):
    ft_ref[0] = jnp.transpose(f_ref[0], (1, 0))
    b = pl.program_id(0)
    x = xyz_ref[0]
    c = cxyz_ref[0]
    xb = x.astype(jnp.bfloat16).astype(jnp.float32)
    cb = c.astype(jnp.bfloat16).astype(jnp.float32)
    x2 = (x[0:1] * x[0:1] + x[1:2] * x[1:2]) + x[2:3] * x[2:3]
    c2 = (c[:, 0:1] * c[:, 0:1] + c[:, 1:2] * c[:, 1:2]) + c[:, 2:3] * c[:, 2:3]
    dot = cb[:, 0:1] * xb[0:1]
    dot = dot + cb[:, 1:2] * xb[1:2]
    dot = dot + cb[:, 2:3] * xb[2:3]
    cur = jnp.maximum((x2 + c2) - 2.0 * dot, 1e-12)
    iota = lax.broadcasted_iota(jnp.int32, (M, TN), 0)
    ds, js = [], []
    for t in range(K):
        m = jnp.min(cur, axis=0, keepdims=True)
        i = jnp.min(jnp.where(cur == m, iota, M), axis=0, keepdims=True)
        ds.append(m)
        js.append(i)
        if t < K - 1:
            cur = jnp.where(iota == i, jnp.float32(3.0e38), cur)
    d = jnp.sqrt(jnp.concatenate(ds, axis=0))
    w = 1.0 / jnp.maximum(d, 1e-8)
    wn = w / jnp.sum(w, axis=0, keepdims=True)
    idx_ref[...] = jnp.concatenate(js, axis=0) + b * M
    w_ref[...] = wn


def _nearest_tc(xyz_t, cxyz, fc, boff):
    nt = N // TN
    pc = CB * N
    return pl.pallas_call(
        functools.partial(_topk_body, boff),
        grid=(CB, nt),
        in_specs=[
            pl.BlockSpec((1, 3, TN), lambda b, n: (b, 0, n)),
            pl.BlockSpec((1, M, 3), lambda b, n: (b, 0, 0)),
            pl.BlockSpec((1, C, M // nt), lambda b, n: (b, 0, n)),
        ],
        out_specs=[
            pl.BlockSpec((K, TN), lambda b, n: (0, b * nt + n)),
            pl.BlockSpec((K, TN), lambda b, n: (0, b * nt + n)),
            pl.BlockSpec((1, M // nt, C), lambda b, n: (b, n, 0)),
        ],
        out_shape=[
            jax.ShapeDtypeStruct((K, pc), jnp.int32),
            jax.ShapeDtypeStruct((K, pc), jnp.float32),
            jax.ShapeDtypeStruct((CB, M, C), jnp.float32),
        ],
    )(xyz_t, cxyz, fc)


def _interp_body(table_hbm, idx_hbm, w_hbm, out_hbm,
                 idx_v, w_v, rows0, rows1, o0, o1,
                 sem_g0, sem_g1, sem_o0, sem_o1):
    pc = CB * N
    pw = pc // NW
    nsub = pw // SUB
    wid = lax.axis_index("s") * NC + lax.axis_index("c")
    base = pl.multiple_of(wid * pw, pw)
    for j in range(K):
        pltpu.sync_copy(idx_hbm.at[pl.ds(j * pc + base, pw)], idx_v.at[pl.ds(j * pw, pw)])
        pltpu.sync_copy(w_hbm.at[pl.ds(j * pc + base, pw)], w_v.at[pl.ds(j * pw, pw)])

    def issue_gather(s, rows, sem):
        off = pl.multiple_of(s * SUB, SUB)
        for j in range(K):
            pltpu.async_copy(
                table_hbm.at[idx_v.at[pl.ds(j * pw + off, SUB)]],
                rows.at[pl.ds(j * SUB, SUB)], sem)

    def wait_gather(rows, sem):
        pltpu.make_async_copy(table_hbm.at[pl.ds(0, K * SUB)], rows, sem).wait()

    def wait_out(o, sem):
        pltpu.make_async_copy(o, out_hbm.at[pl.ds(base, SUB)], sem).wait()

    def issue_out(s, o, sem):
        off = pl.multiple_of(s * SUB, SUB)
        pltpu.async_copy(o, out_hbm.at[pl.ds(base + off, SUB)], sem)

    def compute(s, rows, o):
        off = pl.multiple_of(s * SUB, SUB)
        wvs = [w_v[pl.ds(j * pw + off, L)] for j in range(K)]
        dn = lax.GatherDimensionNumbers(offset_dims=(), collapsed_slice_dims=(0,),
                                        start_index_map=(0,))

        @pl.loop(0, SUB)
        def _pt(pidx):
            lane = jnp.full((L, 1), pidx, jnp.int32)
            w0, w1, w2 = (lax.gather(wv, lane, dn, slice_sizes=(1,),
                                     mode=lax.GatherScatterMode.PROMISE_IN_BOUNDS)
                          for wv in wvs)
            for cc in range(C // L):
                sl = pl.ds(cc * L, L)
                r0 = rows[pidx, sl]
                r1 = rows[SUB + pidx, sl]
                r2 = rows[2 * SUB + pidx, sl]
                o[pidx, sl] = r0 * w0 + r1 * w1 + r2 * w2

    issue_gather(0, rows0, sem_g0)

    @pl.loop(0, nsub, step=2)
    def _ring(s0):
        wait_gather(rows0, sem_g0)
        issue_gather(s0 + 1, rows1, sem_g1)

        @pl.when(s0 >= 2)
        def _():
            wait_out(o0, sem_o0)

        compute(s0, rows0, o0)
        issue_out(s0, o0, sem_o0)

        wait_gather(rows1, sem_g1)

        @pl.when(s0 + 2 < nsub)
        def _():
            issue_gather(s0 + 2, rows0, sem_g0)

        @pl.when(s0 >= 2)
        def _():
            wait_out(o1, sem_o1)

        compute(s0 + 1, rows1, o1)
        issue_out(s0 + 1, o1, sem_o1)

    wait_out(o0, sem_o0)
    wait_out(o1, sem_o1)


@functools.cache
def _interp_sc():
    pc = CB * N
    pw = pc // NW
    return pl.kernel(
        _interp_body,
        out_type=jax.ShapeDtypeStruct((pc, C), jnp.float32),
        mesh=plsc.VectorSubcoreMesh(core_axis_name="c", subcore_axis_name="s",
                                    num_cores=NC, num_subcores=NS),
        scratch_types=[
            pltpu.VMEM((K * pw,), jnp.int32),
            pltpu.VMEM((K * pw,), jnp.float32),
            pltpu.VMEM((K * SUB, C), jnp.float32),
            pltpu.VMEM((K * SUB, C), jnp.float32),
            pltpu.VMEM((SUB, C), jnp.float32),
            pltpu.VMEM((SUB, C), jnp.float32),
            pltpu.SemaphoreType.DMA,
            pltpu.SemaphoreType.DMA,
            pltpu.SemaphoreType.DMA,
            pltpu.SemaphoreType.DMA,
        ],
    )


def kernel(p, f):
    xyz_t = jnp.transpose(p[0], (0, 2, 1))
    cxyz = p[1]
    pc = CB * N
    outs = []
    for ch in range(NCH):
        bs = slice(ch * CB, (ch + 1) * CB)
        idx, w, ft = _nearest_tc(xyz_t[bs], cxyz[bs], f[0][bs], ch * CB)
        outs.append(_interp_sc()(ft.reshape(CB * M, C),
                                 idx.reshape(K * pc), w.reshape(K * pc)))
    return jnp.concatenate(outs, axis=0).reshape(B, N, C)
